# baseline (speedup 1.0000x reference)
import functools

import jax
import jax.numpy as jnp
from jax.experimental import pallas as pl
from jax.experimental.pallas import tpu as pltpu

N = 10000
D = 128
H2 = 128
CLUSTERS = 64
LATENT = 64
BLK = 5000


def _dense_kernel(nodes_ref, w1_ref, b1_ref, w2_ref, b2_ref,
                  wd1_ref, bd1_ref, wd2_ref, bd2_ref,
                  wmu_ref, bmu_ref, wlv_ref, blv_ref,
                  wp1_ref, bp1_ref, wp2_ref, bp2_ref, eps_ref,
                  assign_ref, coarse_ref, mu_ref, lv_ref, py_ref):
    i = pl.program_id(0)
    x = nodes_ref[...]
    h = jnp.dot(x, w1_ref[...], preferred_element_type=jnp.float32) + b1_ref[...]
    h = jnp.maximum(h, 0.0)
    h = jnp.dot(h, w2_ref[...], preferred_element_type=jnp.float32) + b2_ref[...]

    a = jnp.dot(h, wd1_ref[...], preferred_element_type=jnp.float32) + bd1_ref[...]
    a = jnp.maximum(a, 0.0)
    logits = jnp.dot(a, wd2_ref[...], preferred_element_type=jnp.float32) + bd2_ref[...]
    m = jnp.max(logits, axis=-1, keepdims=True)
    e = jnp.exp(logits - m)
    assign = e / jnp.sum(e, axis=-1, keepdims=True)
    assign_ref[...] = assign

    partial = jax.lax.dot_general(assign, h, (((0,), (0,)), ((), ())),
                                  preferred_element_type=jnp.float32)

    @pl.when(i == 0)
    def _():
        coarse_ref[...] = partial

    @pl.when(i > 0)
    def _():
        coarse_ref[...] += partial

    @pl.when(i == pl.num_programs(0) - 1)
    def _():
        coarse = coarse_ref[...]
        macro = jnp.mean(coarse, axis=0, keepdims=True)
        mu = jnp.dot(macro, wmu_ref[...], preferred_element_type=jnp.float32) + bmu_ref[...]
        lv = jnp.dot(macro, wlv_ref[...], preferred_element_type=jnp.float32) + blv_ref[...]
        std = jnp.exp(0.5 * lv)
        z = mu + eps_ref[...] * std
        p = jnp.dot(z, wp1_ref[...], preferred_element_type=jnp.float32) + bp1_ref[...]
        p = jnp.maximum(p, 0.0)
        py = jnp.dot(p, wp2_ref[...], preferred_element_type=jnp.float32) + bp2_ref[...]
        mu_ref[...] = mu
        lv_ref[...] = lv
        py_ref[...] = py


@functools.partial(jax.jit, static_argnames=())
def kernel(nodes, edges, senders, receivers,
           W_enc1, b_enc1, W_enc2, b_enc2,
           W_dec1, b_dec1, W_dec2, b_dec2,
           W_mu, b_mu, W_lv, b_lv,
           W_p1, b_p1, W_p2, b_p2):
    eps = jax.random.normal(jax.random.PRNGKey(0), (LATENT,), jnp.float32)

    row = lambda v: v.reshape(1, -1)
    full = lambda arr: pl.BlockSpec(arr.shape, lambda i: (0, 0))
    grid = N // BLK

    consts = (W_enc1, row(b_enc1), W_enc2, row(b_enc2),
              W_dec1, row(b_dec1), W_dec2, row(b_dec2),
              W_mu, row(b_mu), W_lv, row(b_lv),
              W_p1, row(b_p1), W_p2, row(b_p2), row(eps))

    out = pl.pallas_call(
        _dense_kernel,
        grid=(grid,),
        in_specs=[pl.BlockSpec((BLK, D), lambda i: (i, 0))] + [full(c) for c in consts],
        out_specs=[
            pl.BlockSpec((BLK, CLUSTERS), lambda i: (i, 0)),
            pl.BlockSpec((CLUSTERS, H2), lambda i: (0, 0)),
            pl.BlockSpec((1, LATENT), lambda i: (0, 0)),
            pl.BlockSpec((1, LATENT), lambda i: (0, 0)),
            pl.BlockSpec((1, 1), lambda i: (0, 0)),
        ],
        out_shape=[
            jax.ShapeDtypeStruct((N, CLUSTERS), jnp.float32),
            jax.ShapeDtypeStruct((CLUSTERS, H2), jnp.float32),
            jax.ShapeDtypeStruct((1, LATENT), jnp.float32),
            jax.ShapeDtypeStruct((1, LATENT), jnp.float32),
            jax.ShapeDtypeStruct((1, 1), jnp.float32),
        ],
        compiler_params=pltpu.CompilerParams(
            dimension_semantics=("arbitrary",),
        ),
    )(nodes, *consts)

    assignments, coarse_nodes, mu, lv, py = out
    return (mu.reshape(LATENT), lv.reshape(LATENT), py.reshape(1),
            assignments, coarse_nodes)


# speedup vs baseline: 1.0399x; 1.0399x over previous
import functools

import jax
import jax.numpy as jnp
from jax.experimental import pallas as pl
from jax.experimental.pallas import tpu as pltpu

N = 10000
D = 128
H2 = 128
CLUSTERS = 64
LATENT = 64
BLK = 2000


def _dense_kernel(nodes_ref, w1_ref, b1_ref, w2_ref, b2_ref,
                  wd1_ref, bd1_ref, wd2_ref, bd2_ref,
                  wmu_ref, bmu_ref, wlv_ref, blv_ref,
                  wp1_ref, bp1_ref, wp2_ref, bp2_ref, eps_ref,
                  assign_ref, coarse_ref, mu_ref, lv_ref, py_ref):
    i = pl.program_id(0)
    x = nodes_ref[...]
    h = jnp.dot(x, w1_ref[...], preferred_element_type=jnp.float32) + b1_ref[...]
    h = jnp.maximum(h, 0.0)
    h = jnp.dot(h, w2_ref[...], preferred_element_type=jnp.float32) + b2_ref[...]

    a = jnp.dot(h, wd1_ref[...], preferred_element_type=jnp.float32) + bd1_ref[...]
    a = jnp.maximum(a, 0.0)
    logits = jnp.dot(a, wd2_ref[...], preferred_element_type=jnp.float32) + bd2_ref[...]
    m = jnp.max(logits, axis=-1, keepdims=True)
    e = jnp.exp(logits - m)
    assign = e / jnp.sum(e, axis=-1, keepdims=True)
    assign_ref[...] = assign

    partial = jax.lax.dot_general(assign, h, (((0,), (0,)), ((), ())),
                                  preferred_element_type=jnp.float32)

    @pl.when(i == 0)
    def _():
        coarse_ref[...] = partial

    @pl.when(i > 0)
    def _():
        coarse_ref[...] += partial

    @pl.when(i == pl.num_programs(0) - 1)
    def _():
        coarse = coarse_ref[...]
        macro = jnp.mean(coarse, axis=0, keepdims=True)
        mu = jnp.dot(macro, wmu_ref[...], preferred_element_type=jnp.float32) + bmu_ref[...]
        lv = jnp.dot(macro, wlv_ref[...], preferred_element_type=jnp.float32) + blv_ref[...]
        std = jnp.exp(0.5 * lv)
        z = mu + eps_ref[...] * std
        p = jnp.dot(z, wp1_ref[...], preferred_element_type=jnp.float32) + bp1_ref[...]
        p = jnp.maximum(p, 0.0)
        py = jnp.dot(p, wp2_ref[...], preferred_element_type=jnp.float32) + bp2_ref[...]
        mu_ref[...] = mu
        lv_ref[...] = lv
        py_ref[...] = py


@functools.partial(jax.jit, static_argnames=())
def kernel(nodes, edges, senders, receivers,
           W_enc1, b_enc1, W_enc2, b_enc2,
           W_dec1, b_dec1, W_dec2, b_dec2,
           W_mu, b_mu, W_lv, b_lv,
           W_p1, b_p1, W_p2, b_p2):
    eps = jax.random.normal(jax.random.PRNGKey(0), (LATENT,), jnp.float32)

    row = lambda v: v.reshape(1, -1)
    full = lambda arr: pl.BlockSpec(arr.shape, lambda i: (0, 0))
    grid = N // BLK

    consts = (W_enc1, row(b_enc1), W_enc2, row(b_enc2),
              W_dec1, row(b_dec1), W_dec2, row(b_dec2),
              W_mu, row(b_mu), W_lv, row(b_lv),
              W_p1, row(b_p1), W_p2, row(b_p2), row(eps))

    out = pl.pallas_call(
        _dense_kernel,
        grid=(grid,),
        in_specs=[pl.BlockSpec((BLK, D), lambda i: (i, 0))] + [full(c) for c in consts],
        out_specs=[
            pl.BlockSpec((BLK, CLUSTERS), lambda i: (i, 0)),
            pl.BlockSpec((CLUSTERS, H2), lambda i: (0, 0)),
            pl.BlockSpec((1, LATENT), lambda i: (0, 0)),
            pl.BlockSpec((1, LATENT), lambda i: (0, 0)),
            pl.BlockSpec((1, 1), lambda i: (0, 0)),
        ],
        out_shape=[
            jax.ShapeDtypeStruct((N, CLUSTERS), jnp.float32),
            jax.ShapeDtypeStruct((CLUSTERS, H2), jnp.float32),
            jax.ShapeDtypeStruct((1, LATENT), jnp.float32),
            jax.ShapeDtypeStruct((1, LATENT), jnp.float32),
            jax.ShapeDtypeStruct((1, 1), jnp.float32),
        ],
        compiler_params=pltpu.CompilerParams(
            dimension_semantics=("arbitrary",),
        ),
    )(nodes, *consts)

    assignments, coarse_nodes, mu, lv, py = out
    return (mu.reshape(LATENT), lv.reshape(LATENT), py.reshape(1),
            assignments, coarse_nodes)
